# Initial kernel scaffold; baseline (speedup 1.0000x reference)
import jax
import jax.numpy as jnp
from jax.experimental import pallas as pl


def kernel(x, node_type_ids, edge_index, batch, emb_table, W_proj, b_proj, W_gcn, b_gcn, W_cls, b_cls):
    raise NotImplementedError("write your pallas kernel here")


# speedup vs baseline: 46.3354x; 46.3354x over previous
import jax
import jax.numpy as jnp
from jax import lax
from jax.experimental import pallas as pl
from jax.experimental.pallas import tpu as pltpu
from jax.experimental.pallas import tpu_sc as plsc


N = 100000
E = 1600000
IN_DIM = 128
EMB = 32
HALF = 16
VOCAB = 64
NUM_GRAPHS = 64
NUM_CLASSES = 4

CHUNK = 128
SS = 4
NSUP = E // (SS * CHUNK)
NSUB = 16
NSC = 2

N_PAD = -(-N // 128) * 128
PER16 = (N_PAD // (128 * NSUB)) * 128
TAIL16 = N_PAD - NSUB * PER16
NBLK_FULL = N // 128


def _mesh():
  return plsc.VectorSubcoreMesh(core_axis_name="c", subcore_axis_name="s")


_SC_PARAMS = pltpu.CompilerParams(use_tc_tiling_on_sc=False,
                                  needs_layout_passes=False)


def _fill_1d(ref, val):
  n = ref.shape[0]
  v = jnp.full((HALF,), val, ref.dtype)
  for i in range(n // HALF):
    ref[pl.ds(i * HALF, HALF)] = v


def _fill_2d(ref, val):
  v = jnp.full((HALF,), val, ref.dtype)
  for i in range(ref.shape[0]):
    ref[i] = v


def _zero_slice(zbuf, dst, lo, is_last, tail):
  full = zbuf.shape[0]
  n_full, rem = divmod(PER16, full)
  for i in range(n_full):
    pltpu.sync_copy(zbuf, dst.at[pl.ds(lo + i * full, full)])
  if rem:
    pltpu.sync_copy(zbuf.at[pl.ds(0, rem)],
                    dst.at[pl.ds(lo + n_full * full, rem)])

  @pl.when(is_last)
  def _():
    base = NSUB * PER16
    n_t, rem_t = divmod(tail, full)
    for i in range(n_t):
      pltpu.sync_copy(zbuf, dst.at[pl.ds(base + i * full, full)])
    if rem_t:
      pltpu.sync_copy(zbuf.at[pl.ds(0, rem_t)],
                      dst.at[pl.ds(base + n_t * full, rem_t)])


def _copy_slice(src, dst, lo, is_last, tail):
  pltpu.sync_copy(src.at[pl.ds(lo, PER16)], dst.at[pl.ds(lo, PER16)])

  @pl.when(is_last)
  def _():
    pltpu.sync_copy(src.at[pl.ds(NSUB * PER16, tail)],
                    dst.at[pl.ds(NSUB * PER16, tail)])


def _deg_kernel_body(e4_hbm, degp_hbm, idx_v, ones_v, zbuf_v, acc_sh,
                     isem, ssem):
  c = lax.axis_index("c")
  s = lax.axis_index("s")
  w = s * NSC + c

  _fill_1d(zbuf_v, 0.0)
  lo = s * PER16
  _zero_slice(zbuf_v, acc_sh, lo, s == NSUB - 1, TAIL16)
  _fill_1d(ones_v, 1.0)
  plsc.subcore_barrier()

  sl = (NSUP * w) // 32
  sh = (NSUP * (w + 1)) // 32
  nsup = 100

  def fire_idx(k, q):
    @pl.when((k >= 0) & (sl + k < sh))
    def _():
      pltpu.async_copy(e4_hbm.at[1, sl + k], idx_v.at[q], isem)

  def wait_idx(k, q):
    @pl.when((k >= 0) & (sl + k < sh))
    def _():
      pltpu.make_async_copy(e4_hbm.at[1, sl + k], idx_v.at[q], isem).wait()

  def fire_scatter(k, q):
    @pl.when((k >= 0) & (sl + k < sh))
    def _():
      for j in range(SS):
        pltpu.async_copy(ones_v, acc_sh.at[idx_v.at[q, j]], ssem, add=True)

  def wait_scatter(k, q):
    @pl.when((k >= 0) & (sl + k < sh))
    def _():
      for j in range(SS):
        pltpu.make_async_copy(ones_v, acc_sh.at[idx_v.at[q, j]], ssem).wait()

  fire_idx(0, 0)

  def sbody(kk, _):
    for u in range(4):
      k = kk * 4 + u
      wait_idx(k, u)
      wait_scatter(k - 2, (u - 2) % 4)
      fire_idx(k + 1, (u + 1) % 4)
      fire_scatter(k, u)
    return 0

  lax.fori_loop(0, nsup // 4, sbody, 0)
  wait_scatter(nsup - 2, (nsup - 2) % 4)
  wait_scatter(nsup - 1, (nsup - 1) % 4)
  plsc.subcore_barrier()

  _copy_slice(acc_sh, degp_hbm.at[c], lo, s == NSUB - 1, TAIL16)


def _deg_call(e4):
  kern = pl.kernel(
      _deg_kernel_body,
      out_type=jax.ShapeDtypeStruct((NSC, N_PAD), jnp.float32),
      mesh=_mesh(),
      compiler_params=_SC_PARAMS,
      scratch_types=[
          pltpu.VMEM((4, SS, CHUNK), jnp.int32),
          pltpu.VMEM((CHUNK,), jnp.float32),
          pltpu.VMEM((1024,), jnp.float32),
          pltpu.VMEM_SHARED((N_PAD,), jnp.float32),
          pltpu.SemaphoreType.DMA,
          pltpu.SemaphoreType.DMA,
      ],
  )
  return kern(e4)


def _proj_body(ids_ref, x_ref, degp_ref, emb_ref, wp_ref, bp_ref, wg_ref,
               g_ref):
  wg = wg_ref[...]
  a = jnp.dot(wp_ref[:IN_DIM, :], wg, preferred_element_type=jnp.float32)
  lut = jnp.dot(
      jnp.dot(emb_ref[...], wp_ref[IN_DIM:, :],
              preferred_element_type=jnp.float32) + bp_ref[...][None, :],
      wg, preferred_element_type=jnp.float32)
  ids = ids_ref[0]
  onehot = (ids == lax.broadcasted_iota(jnp.int32, (1, VOCAB), 1)
            ).astype(jnp.float32)
  hg = (jnp.dot(x_ref[...], a, preferred_element_type=jnp.float32)
        + jnp.dot(onehot, lut, preferred_element_type=jnp.float32))
  deg = degp_ref[0, 0] + degp_ref[1, 0] + 1.0
  g = hg * lax.rsqrt(deg)
  g_ref[0] = g[:, :HALF]
  g_ref[1] = g[:, HALF:]


def _proj_call(ids3, x, degp4, emb, wp, bp, wg):
  blk = 4000
  grid = N // blk
  return pl.pallas_call(
      _proj_body,
      grid=(grid,),
      in_specs=[
          pl.BlockSpec((1, blk, 1), lambda i: (i, 0, 0)),
          pl.BlockSpec((blk, IN_DIM), lambda i: (i, 0)),
          pl.BlockSpec((2, 1, blk, 1), lambda i: (0, i, 0, 0)),
          pl.BlockSpec((VOCAB, EMB), lambda i: (0, 0)),
          pl.BlockSpec((IN_DIM + EMB, EMB), lambda i: (0, 0)),
          pl.BlockSpec((EMB,), lambda i: (0,)),
          pl.BlockSpec((EMB, EMB), lambda i: (0, 0)),
      ],
      out_specs=pl.BlockSpec((2, blk, HALF), lambda i: (0, i, 0)),
      out_shape=jax.ShapeDtypeStruct((2, N, HALF), jnp.float32),
  )(ids3, x, degp4, emb, wp, bp, wg)


def _agg_kernel_body(e4_hbm, g_hbm, acc_hbm,
                     idxs_v, idxd_v, rows_v, zbuf_v, acc_sh,
                     isem, gsem, ssem):
  c = lax.axis_index("c")
  s = lax.axis_index("s")

  _fill_2d(zbuf_v, 0.0)
  lo = s * PER16
  _zero_slice(zbuf_v, acc_sh, lo, s == NSUB - 1, N - NSUB * PER16)
  plsc.subcore_barrier()

  gsrc = g_hbm.at[c]
  sl = (NSUP * s) // NSUB
  sh = (NSUP * (s + 1)) // NSUB
  nsup = 196

  def fire_idx(k, q):
    @pl.when((k >= 0) & (sl + k < sh))
    def _():
      pltpu.async_copy(e4_hbm.at[0, sl + k], idxs_v.at[q], isem)
      pltpu.async_copy(e4_hbm.at[1, sl + k], idxd_v.at[q], isem)

  def wait_idx(k, q):
    @pl.when((k >= 0) & (sl + k < sh))
    def _():
      pltpu.make_async_copy(e4_hbm.at[0, sl + k], idxs_v.at[q], isem).wait()
      pltpu.make_async_copy(e4_hbm.at[1, sl + k], idxd_v.at[q], isem).wait()

  def fire_gather(k, p, q):
    @pl.when((k >= 0) & (sl + k < sh))
    def _():
      for j in range(SS):
        pltpu.async_copy(gsrc.at[idxs_v.at[q, j]], rows_v.at[p, j], gsem)

  def wait_gather(k, p, q):
    @pl.when((k >= 0) & (sl + k < sh))
    def _():
      for j in range(SS):
        pltpu.make_async_copy(gsrc.at[idxs_v.at[q, j]], rows_v.at[p, j],
                              gsem).wait()

  def fire_scatter(k, p, q):
    @pl.when((k >= 0) & (sl + k < sh))
    def _():
      for j in range(SS):
        pltpu.async_copy(rows_v.at[p, j], acc_sh.at[idxd_v.at[q, j]], ssem,
                         add=True)

  def wait_scatter(k, p, q):
    @pl.when((k >= 0) & (sl + k < sh))
    def _():
      for j in range(SS):
        pltpu.make_async_copy(rows_v.at[p, j], acc_sh.at[idxd_v.at[q, j]],
                              ssem).wait()

  fire_idx(0, 0)

  def sbody(kk, _):
    for u in range(4):
      k = kk * 4 + u
      p = u % 2
      wait_idx(k, u)
      wait_scatter(k - 2, p, (u - 2) % 4)
      fire_gather(k, p, u)
      fire_idx(k + 1, (u + 1) % 4)
      wait_gather(k, p, u)
      fire_scatter(k, p, u)
    return 0

  lax.fori_loop(0, nsup // 4, sbody, 0)
  wait_scatter(nsup - 2, 0, (nsup - 2) % 4)
  wait_scatter(nsup - 1, 1, (nsup - 1) % 4)
  plsc.subcore_barrier()

  _copy_slice(acc_sh, acc_hbm.at[c], lo, s == NSUB - 1, N - NSUB * PER16)


def _agg_call(e4, g):
  kern = pl.kernel(
      _agg_kernel_body,
      out_type=jax.ShapeDtypeStruct((NSC, N, HALF), jnp.float32),
      mesh=_mesh(),
      compiler_params=_SC_PARAMS,
      scratch_types=[
          pltpu.VMEM((4, SS, CHUNK), jnp.int32),
          pltpu.VMEM((4, SS, CHUNK), jnp.int32),
          pltpu.VMEM((2, SS, CHUNK, HALF), jnp.float32),
          pltpu.VMEM((128, HALF), jnp.float32),
          pltpu.VMEM_SHARED((N, HALF), jnp.float32),
          pltpu.SemaphoreType.DMA,
          pltpu.SemaphoreType.DMA,
          pltpu.SemaphoreType.DMA,
      ],
  )
  return kern(e4, g)


def _epi_kernel_body(acc_hbm, g_hbm, degp_hbm, batch_hbm, bgcn_hbm,
                     partials_hbm,
                     a0_v, a1_v, q0_v, q1_v, p0_v, p1_v, bt_v, bias_v,
                     seg_v):
  c = lax.axis_index("c")
  s = lax.axis_index("s")
  w = s * NSC + c
  cb = a0_v.shape[0]

  pltpu.sync_copy(bgcn_hbm, bias_v)
  b0 = bias_v[pl.ds(0, HALF)]
  b1 = bias_v[pl.ds(HALF, HALF)]

  zrow = jnp.zeros((HALF,), jnp.float32)
  for gi in range((NUM_GRAPHS + 1) * EMB // HALF):
    seg_v[pl.ds(gi * HALF, HALF)] = zrow

  lane = lax.broadcasted_iota(jnp.int32, (HALF,), 0)

  def do_chunk(base, size, fsize):
    pltpu.sync_copy(acc_hbm.at[0, pl.ds(base, size)], a0_v.at[pl.ds(0, size)])
    pltpu.sync_copy(acc_hbm.at[1, pl.ds(base, size)], a1_v.at[pl.ds(0, size)])
    pltpu.sync_copy(g_hbm.at[0, pl.ds(base, size)], q0_v.at[pl.ds(0, size)])
    pltpu.sync_copy(g_hbm.at[1, pl.ds(base, size)], q1_v.at[pl.ds(0, size)])
    pltpu.sync_copy(degp_hbm.at[0, pl.ds(base, fsize)],
                    p0_v.at[pl.ds(0, fsize)])
    pltpu.sync_copy(degp_hbm.at[1, pl.ds(base, fsize)],
                    p1_v.at[pl.ds(0, fsize)])
    pltpu.sync_copy(batch_hbm.at[pl.ds(base, fsize)], bt_v.at[pl.ds(0, fsize)])

    def group_body(gi, _):
      r0 = gi * HALF
      deg = p0_v[pl.ds(r0, HALF)] + p1_v[pl.ds(r0, HALF)] + 1.0
      ib = plsc.bitcast(deg, jnp.int32)
      y = plsc.bitcast(0x5F3759DF - lax.shift_right_arithmetic(ib, 1),
                       jnp.float32)
      for _ in range(3):
        y = y * (1.5 - 0.5 * deg * y * y)
      bt = bt_v[pl.ds(r0, HALF)]
      for j in range(HALF):
        r = r0 + j
        dj = y[j]
        base_ix = bt[j] * EMB + lane
        h0 = jnp.maximum((a0_v[r] + q0_v[r]) * dj + b0, 0.0)
        h1 = jnp.maximum((a1_v[r] + q1_v[r]) * dj + b1, 0.0)
        m0 = plsc.load_gather(seg_v, [base_ix])
        plsc.store_scatter(seg_v, [base_ix], jnp.maximum(m0, h0))
        m1 = plsc.load_gather(seg_v, [base_ix + HALF])
        plsc.store_scatter(seg_v, [base_ix + HALF], jnp.maximum(m1, h1))
      return 0

    lax.fori_loop(0, size // HALF, group_body, 0)

  lo_b = (NBLK_FULL * w) // 32
  hi_b = (NBLK_FULL * (w + 1)) // 32
  base = lo_b * 128
  span = (hi_b - lo_b) * 128
  n_cb = span // cb

  def cbody(i, _):
    do_chunk(base + i * cb, cb, cb)
    return 0

  lax.fori_loop(0, n_cb, cbody, 0)
  rem = span - n_cb * cb
  base2 = base + n_cb * cb
  n128 = rem // 128

  def cbody128(i, _):
    do_chunk(base2 + i * 128, 128, 128)
    return 0

  lax.fori_loop(0, n128, cbody128, 0)

  @pl.when(w == 31)
  def _():
    do_chunk(NBLK_FULL * 128, N - NBLK_FULL * 128, 128)

  pltpu.sync_copy(seg_v, partials_hbm.at[w, 0])


def _epi_call(acc, g, degp, batch, b_gcn):
  cb = 512
  kern = pl.kernel(
      _epi_kernel_body,
      out_type=jax.ShapeDtypeStruct((32, 1, (NUM_GRAPHS + 1) * EMB),
                                    jnp.float32),
      mesh=_mesh(),
      compiler_params=_SC_PARAMS,
      scratch_types=[
          pltpu.VMEM((cb, HALF), jnp.float32),
          pltpu.VMEM((cb, HALF), jnp.float32),
          pltpu.VMEM((cb, HALF), jnp.float32),
          pltpu.VMEM((cb, HALF), jnp.float32),
          pltpu.VMEM((cb,), jnp.float32),
          pltpu.VMEM((cb,), jnp.float32),
          pltpu.VMEM((cb,), jnp.int32),
          pltpu.VMEM((128,), jnp.float32),
          pltpu.VMEM(((NUM_GRAPHS + 1) * EMB,), jnp.float32),
      ],
  )
  return kern(acc, g, degp, batch, b_gcn)


def _head_body(part_ref, wc_ref, bc_ref, out_ref):
  rep = jnp.max(part_ref[...], axis=0)
  out_ref[...] = (jnp.dot(rep, wc_ref[...],
                          preferred_element_type=jnp.float32)
                  + bc_ref[...][None, :])


def _head_call(partials3, w_cls, b_cls):
  return pl.pallas_call(
      _head_body,
      out_shape=jax.ShapeDtypeStruct((NUM_GRAPHS, NUM_CLASSES), jnp.float32),
  )(partials3, w_cls, b_cls)


def kernel(x, node_type_ids, edge_index, batch, emb_table, W_proj, b_proj,
           W_gcn, b_gcn, W_cls, b_cls):
  e4 = edge_index.reshape(2, NSUP, SS, CHUNK)
  degp = _deg_call(e4)
  ids3 = node_type_ids.reshape(N // 4000, 4000, 1)
  degp4 = degp[:, :N].reshape(2, N // 4000, 4000, 1)
  g = _proj_call(ids3, x, degp4, emb_table, W_proj, b_proj, W_gcn)
  acc = _agg_call(e4, g)
  batch_pad = jnp.concatenate(
      [batch, jnp.full((N_PAD - N,), NUM_GRAPHS, jnp.int32)])
  bgcn_pad = jnp.pad(b_gcn, (0, 128 - EMB))
  partials = _epi_call(acc, g, degp, batch_pad, bgcn_pad)
  partials3 = partials.reshape(32, NUM_GRAPHS + 1, EMB)[:, :NUM_GRAPHS, :]
  return _head_call(partials3, W_cls, b_cls)


# speedup vs baseline: 49.9356x; 1.0777x over previous
import jax
import jax.numpy as jnp
from jax import lax
from jax.experimental import pallas as pl
from jax.experimental.pallas import tpu as pltpu
from jax.experimental.pallas import tpu_sc as plsc


N = 100000
E = 1600000
IN_DIM = 128
EMB = 32
HALF = 16
VOCAB = 64
NUM_GRAPHS = 64
NUM_CLASSES = 4

CHUNK = 512
NSUP = E // CHUNK
NSUB = 16
NSC = 2

N_PAD = -(-N // 128) * 128
PER16 = (N_PAD // (128 * NSUB)) * 128
TAIL16 = N_PAD - NSUB * PER16
NBLK_FULL = N // 128


def _mesh():
  return plsc.VectorSubcoreMesh(core_axis_name="c", subcore_axis_name="s")


_SC_PARAMS = pltpu.CompilerParams(use_tc_tiling_on_sc=False,
                                  needs_layout_passes=False)


def _fill_1d(ref, val):
  n = ref.shape[0]
  v = jnp.full((HALF,), val, ref.dtype)
  for i in range(n // HALF):
    ref[pl.ds(i * HALF, HALF)] = v


def _fill_2d(ref, val):
  v = jnp.full((HALF,), val, ref.dtype)
  for i in range(ref.shape[0]):
    ref[i] = v


def _zero_slice(zbuf, dst, lo, is_last, tail):
  full = zbuf.shape[0]
  n_full, rem = divmod(PER16, full)
  for i in range(n_full):
    pltpu.sync_copy(zbuf, dst.at[pl.ds(lo + i * full, full)])
  if rem:
    pltpu.sync_copy(zbuf.at[pl.ds(0, rem)],
                    dst.at[pl.ds(lo + n_full * full, rem)])

  @pl.when(is_last)
  def _():
    base = NSUB * PER16
    n_t, rem_t = divmod(tail, full)
    for i in range(n_t):
      pltpu.sync_copy(zbuf, dst.at[pl.ds(base + i * full, full)])
    if rem_t:
      pltpu.sync_copy(zbuf.at[pl.ds(0, rem_t)],
                      dst.at[pl.ds(base + n_t * full, rem_t)])


def _copy_slice(src, dst, lo, is_last, tail):
  pltpu.sync_copy(src.at[pl.ds(lo, PER16)], dst.at[pl.ds(lo, PER16)])

  @pl.when(is_last)
  def _():
    pltpu.sync_copy(src.at[pl.ds(NSUB * PER16, tail)],
                    dst.at[pl.ds(NSUB * PER16, tail)])


def _deg_kernel_body(e4_hbm, degp_hbm, idx_v, ones_v, zbuf_v, acc_sh,
                     isem, ssem):
  c = lax.axis_index("c")
  s = lax.axis_index("s")
  w = s * NSC + c

  _fill_1d(zbuf_v, 0.0)
  lo = s * PER16
  _zero_slice(zbuf_v, acc_sh, lo, s == NSUB - 1, TAIL16)
  _fill_1d(ones_v, 1.0)
  plsc.subcore_barrier()

  sl = (NSUP * w) // 32
  sh = (NSUP * (w + 1)) // 32
  nsup = 100

  def fire_idx(k, q):
    @pl.when((k >= 0) & (sl + k < sh))
    def _():
      pltpu.async_copy(e4_hbm.at[1, sl + k], idx_v.at[q], isem)

  def wait_idx(k, q):
    @pl.when((k >= 0) & (sl + k < sh))
    def _():
      pltpu.make_async_copy(e4_hbm.at[1, sl + k], idx_v.at[q], isem).wait()

  def fire_scatter(k, q):
    @pl.when((k >= 0) & (sl + k < sh))
    def _():
      pltpu.async_copy(ones_v, acc_sh.at[idx_v.at[q, 0]], ssem, add=True)

  def wait_scatter(k, q):
    @pl.when((k >= 0) & (sl + k < sh))
    def _():
      pltpu.make_async_copy(ones_v, acc_sh.at[idx_v.at[q, 0]], ssem).wait()

  fire_idx(0, 0)

  def sbody(kk, _):
    for u in range(4):
      k = kk * 4 + u
      wait_idx(k, u)
      wait_scatter(k - 2, (u - 2) % 4)
      fire_idx(k + 1, (u + 1) % 4)
      fire_scatter(k, u)
    return 0

  lax.fori_loop(0, nsup // 4, sbody, 0)
  wait_scatter(nsup - 2, (nsup - 2) % 4)
  wait_scatter(nsup - 1, (nsup - 1) % 4)
  plsc.subcore_barrier()

  _copy_slice(acc_sh, degp_hbm.at[c], lo, s == NSUB - 1, TAIL16)


def _deg_call(e4):
  kern = pl.kernel(
      _deg_kernel_body,
      out_type=jax.ShapeDtypeStruct((NSC, N_PAD), jnp.float32),
      mesh=_mesh(),
      compiler_params=_SC_PARAMS,
      scratch_types=[
          pltpu.VMEM((4, 1, CHUNK), jnp.int32),
          pltpu.VMEM((CHUNK,), jnp.float32),
          pltpu.VMEM((1024,), jnp.float32),
          pltpu.VMEM_SHARED((N_PAD,), jnp.float32),
          pltpu.SemaphoreType.DMA,
          pltpu.SemaphoreType.DMA,
      ],
  )
  return kern(e4)


def _proj_body(ids_ref, x_ref, degp_ref, emb_ref, wp_ref, bp_ref, wg_ref,
               g_ref):
  wg = wg_ref[...]
  a = jnp.dot(wp_ref[:IN_DIM, :], wg, preferred_element_type=jnp.float32)
  lut = jnp.dot(
      jnp.dot(emb_ref[...], wp_ref[IN_DIM:, :],
              preferred_element_type=jnp.float32) + bp_ref[...][None, :],
      wg, preferred_element_type=jnp.float32)
  ids = ids_ref[0]
  onehot = (ids == lax.broadcasted_iota(jnp.int32, (1, VOCAB), 1)
            ).astype(jnp.float32)
  hg = (jnp.dot(x_ref[...], a, preferred_element_type=jnp.float32)
        + jnp.dot(onehot, lut, preferred_element_type=jnp.float32))
  deg = degp_ref[0, 0] + degp_ref[1, 0] + 1.0
  g = hg * lax.rsqrt(deg)
  g_ref[0] = g[:, :HALF]
  g_ref[1] = g[:, HALF:]


def _proj_call(ids3, x, degp4, emb, wp, bp, wg):
  blk = 4000
  grid = N // blk
  return pl.pallas_call(
      _proj_body,
      grid=(grid,),
      in_specs=[
          pl.BlockSpec((1, blk, 1), lambda i: (i, 0, 0)),
          pl.BlockSpec((blk, IN_DIM), lambda i: (i, 0)),
          pl.BlockSpec((2, 1, blk, 1), lambda i: (0, i, 0, 0)),
          pl.BlockSpec((VOCAB, EMB), lambda i: (0, 0)),
          pl.BlockSpec((IN_DIM + EMB, EMB), lambda i: (0, 0)),
          pl.BlockSpec((EMB,), lambda i: (0,)),
          pl.BlockSpec((EMB, EMB), lambda i: (0, 0)),
      ],
      out_specs=pl.BlockSpec((2, blk, HALF), lambda i: (0, i, 0)),
      out_shape=jax.ShapeDtypeStruct((2, N, HALF), jnp.float32),
  )(ids3, x, degp4, emb, wp, bp, wg)


def _agg_kernel_body(e4_hbm, g_hbm, acc_hbm,
                     idxs_v, idxd_v, rows_v, zbuf_v, acc_sh,
                     isem, gsem, ssem):
  c = lax.axis_index("c")
  s = lax.axis_index("s")

  _fill_2d(zbuf_v, 0.0)
  lo = s * PER16
  _zero_slice(zbuf_v, acc_sh, lo, s == NSUB - 1, N - NSUB * PER16)
  plsc.subcore_barrier()

  gsrc = g_hbm.at[c]
  sl = (NSUP * s) // NSUB
  sh = (NSUP * (s + 1)) // NSUB
  nsup = 204

  def fire_idx(k, q):
    @pl.when((k >= 0) & (sl + k < sh))
    def _():
      pltpu.async_copy(e4_hbm.at[0, sl + k], idxs_v.at[q], isem)
      pltpu.async_copy(e4_hbm.at[1, sl + k], idxd_v.at[q], isem)

  def wait_idx(k, q):
    @pl.when((k >= 0) & (sl + k < sh))
    def _():
      pltpu.make_async_copy(e4_hbm.at[0, sl + k], idxs_v.at[q], isem).wait()
      pltpu.make_async_copy(e4_hbm.at[1, sl + k], idxd_v.at[q], isem).wait()

  def fire_gather(k, p, q):
    @pl.when((k >= 0) & (sl + k < sh))
    def _():
      pltpu.async_copy(gsrc.at[idxs_v.at[q, 0]], rows_v.at[p], gsem)

  def wait_gather(k, p, q):
    @pl.when((k >= 0) & (sl + k < sh))
    def _():
      pltpu.make_async_copy(gsrc.at[idxs_v.at[q, 0]], rows_v.at[p],
                            gsem).wait()

  def fire_scatter(k, p, q):
    @pl.when((k >= 0) & (sl + k < sh))
    def _():
      pltpu.async_copy(rows_v.at[p], acc_sh.at[idxd_v.at[q, 0]], ssem,
                       add=True)

  def wait_scatter(k, p, q):
    @pl.when((k >= 0) & (sl + k < sh))
    def _():
      pltpu.make_async_copy(rows_v.at[p], acc_sh.at[idxd_v.at[q, 0]],
                            ssem).wait()

  fire_idx(0, 0)
  fire_idx(1, 1)
  wait_idx(0, 0)
  fire_gather(0, 0, 0)

  def sbody(kk, _):
    for u in range(12):
      k = kk * 12 + u
      p = u % 3
      q = u % 4
      wait_idx(k + 1, (u + 1) % 4)
      wait_scatter(k - 2, (u - 2) % 3, (u - 2) % 4)
      fire_gather(k + 1, (u + 1) % 3, (u + 1) % 4)
      fire_idx(k + 2, (u + 2) % 4)
      wait_gather(k, p, q)
      fire_scatter(k, p, q)
    return 0

  lax.fori_loop(0, nsup // 12, sbody, 0)
  wait_scatter(nsup - 2, (nsup - 2) % 3, (nsup - 2) % 4)
  wait_scatter(nsup - 1, (nsup - 1) % 3, (nsup - 1) % 4)
  plsc.subcore_barrier()

  _copy_slice(acc_sh, acc_hbm.at[c], lo, s == NSUB - 1, N - NSUB * PER16)


def _agg_call(e4, g):
  kern = pl.kernel(
      _agg_kernel_body,
      out_type=jax.ShapeDtypeStruct((NSC, N, HALF), jnp.float32),
      mesh=_mesh(),
      compiler_params=_SC_PARAMS,
      scratch_types=[
          pltpu.VMEM((4, 1, CHUNK), jnp.int32),
          pltpu.VMEM((4, 1, CHUNK), jnp.int32),
          pltpu.VMEM((3, CHUNK, HALF), jnp.float32),
          pltpu.VMEM((64, HALF), jnp.float32),
          pltpu.VMEM_SHARED((N, HALF), jnp.float32),
          pltpu.SemaphoreType.DMA,
          pltpu.SemaphoreType.DMA,
          pltpu.SemaphoreType.DMA,
      ],
  )
  return kern(e4, g)


def _epi_kernel_body(acc_hbm, g_hbm, degp_hbm, batch_hbm, bgcn_hbm,
                     partials_hbm,
                     a0_v, a1_v, q0_v, q1_v, p0_v, p1_v, bt_v, bias_v,
                     seg_v):
  c = lax.axis_index("c")
  s = lax.axis_index("s")
  w = s * NSC + c
  cb = a0_v.shape[0]

  pltpu.sync_copy(bgcn_hbm, bias_v)
  b0 = bias_v[pl.ds(0, HALF)]
  b1 = bias_v[pl.ds(HALF, HALF)]

  zrow = jnp.zeros((HALF,), jnp.float32)
  for gi in range((NUM_GRAPHS + 1) * EMB // HALF):
    seg_v[pl.ds(gi * HALF, HALF)] = zrow

  lane = lax.broadcasted_iota(jnp.int32, (HALF,), 0)

  def do_chunk(base, size, fsize):
    pltpu.sync_copy(acc_hbm.at[0, pl.ds(base, size)], a0_v.at[pl.ds(0, size)])
    pltpu.sync_copy(acc_hbm.at[1, pl.ds(base, size)], a1_v.at[pl.ds(0, size)])
    pltpu.sync_copy(g_hbm.at[0, pl.ds(base, size)], q0_v.at[pl.ds(0, size)])
    pltpu.sync_copy(g_hbm.at[1, pl.ds(base, size)], q1_v.at[pl.ds(0, size)])
    pltpu.sync_copy(degp_hbm.at[0, pl.ds(base, fsize)],
                    p0_v.at[pl.ds(0, fsize)])
    pltpu.sync_copy(degp_hbm.at[1, pl.ds(base, fsize)],
                    p1_v.at[pl.ds(0, fsize)])
    pltpu.sync_copy(batch_hbm.at[pl.ds(base, fsize)], bt_v.at[pl.ds(0, fsize)])

    def group_body(gi, _):
      r0 = gi * HALF
      deg = p0_v[pl.ds(r0, HALF)] + p1_v[pl.ds(r0, HALF)] + 1.0
      ib = plsc.bitcast(deg, jnp.int32)
      y = plsc.bitcast(0x5F3759DF - lax.shift_right_arithmetic(ib, 1),
                       jnp.float32)
      for _ in range(3):
        y = y * (1.5 - 0.5 * deg * y * y)
      bt = bt_v[pl.ds(r0, HALF)]
      for j in range(HALF):
        r = r0 + j
        dj = y[j]
        base_ix = bt[j] * EMB + lane
        h0 = jnp.maximum((a0_v[r] + q0_v[r]) * dj + b0, 0.0)
        h1 = jnp.maximum((a1_v[r] + q1_v[r]) * dj + b1, 0.0)
        m0 = plsc.load_gather(seg_v, [base_ix])
        plsc.store_scatter(seg_v, [base_ix], jnp.maximum(m0, h0))
        m1 = plsc.load_gather(seg_v, [base_ix + HALF])
        plsc.store_scatter(seg_v, [base_ix + HALF], jnp.maximum(m1, h1))
      return 0

    lax.fori_loop(0, size // HALF, group_body, 0)

  lo_b = (NBLK_FULL * w) // 32
  hi_b = (NBLK_FULL * (w + 1)) // 32
  base = lo_b * 128
  span = (hi_b - lo_b) * 128
  n_cb = span // cb

  def cbody(i, _):
    do_chunk(base + i * cb, cb, cb)
    return 0

  lax.fori_loop(0, n_cb, cbody, 0)
  rem = span - n_cb * cb
  base2 = base + n_cb * cb
  n128 = rem // 128

  def cbody128(i, _):
    do_chunk(base2 + i * 128, 128, 128)
    return 0

  lax.fori_loop(0, n128, cbody128, 0)

  @pl.when(w == 31)
  def _():
    do_chunk(NBLK_FULL * 128, N - NBLK_FULL * 128, 128)

  pltpu.sync_copy(seg_v, partials_hbm.at[w, 0])


def _epi_call(acc, g, degp, batch, b_gcn):
  cb = 512
  kern = pl.kernel(
      _epi_kernel_body,
      out_type=jax.ShapeDtypeStruct((32, 1, (NUM_GRAPHS + 1) * EMB),
                                    jnp.float32),
      mesh=_mesh(),
      compiler_params=_SC_PARAMS,
      scratch_types=[
          pltpu.VMEM((cb, HALF), jnp.float32),
          pltpu.VMEM((cb, HALF), jnp.float32),
          pltpu.VMEM((cb, HALF), jnp.float32),
          pltpu.VMEM((cb, HALF), jnp.float32),
          pltpu.VMEM((cb,), jnp.float32),
          pltpu.VMEM((cb,), jnp.float32),
          pltpu.VMEM((cb,), jnp.int32),
          pltpu.VMEM((128,), jnp.float32),
          pltpu.VMEM(((NUM_GRAPHS + 1) * EMB,), jnp.float32),
      ],
  )
  return kern(acc, g, degp, batch, b_gcn)


def _head_body(part_ref, wc_ref, bc_ref, out_ref):
  rep = jnp.max(part_ref[...], axis=0)
  out_ref[...] = (jnp.dot(rep, wc_ref[...],
                          preferred_element_type=jnp.float32)
                  + bc_ref[...][None, :])


def _head_call(partials3, w_cls, b_cls):
  return pl.pallas_call(
      _head_body,
      out_shape=jax.ShapeDtypeStruct((NUM_GRAPHS, NUM_CLASSES), jnp.float32),
  )(partials3, w_cls, b_cls)


def kernel(x, node_type_ids, edge_index, batch, emb_table, W_proj, b_proj,
           W_gcn, b_gcn, W_cls, b_cls):
  e4 = edge_index.reshape(2, NSUP, 1, CHUNK)
  degp = _deg_call(e4)
  ids3 = node_type_ids.reshape(N // 4000, 4000, 1)
  degp4 = degp[:, :N].reshape(2, N // 4000, 4000, 1)
  g = _proj_call(ids3, x, degp4, emb_table, W_proj, b_proj, W_gcn)
  acc = _agg_call(e4, g)
  batch_pad = jnp.concatenate(
      [batch, jnp.full((N_PAD - N,), NUM_GRAPHS, jnp.int32)])
  bgcn_pad = jnp.pad(b_gcn, (0, 128 - EMB))
  partials = _epi_call(acc, g, degp, batch_pad, bgcn_pad)
  partials3 = partials.reshape(32, NUM_GRAPHS + 1, EMB)[:, :NUM_GRAPHS, :]
  return _head_call(partials3, W_cls, b_cls)


# speedup vs baseline: 51.1004x; 1.0233x over previous
import jax
import jax.numpy as jnp
from jax import lax
from jax.experimental import pallas as pl
from jax.experimental.pallas import tpu as pltpu
from jax.experimental.pallas import tpu_sc as plsc


N = 100000
E = 1600000
IN_DIM = 128
EMB = 32
HALF = 16
VOCAB = 64
NUM_GRAPHS = 64
NUM_CLASSES = 4

CHUNK = 512
NSUP = E // CHUNK
NSUB = 16
NSC = 2

N_PAD = -(-N // 128) * 128
PER16 = (N_PAD // (128 * NSUB)) * 128
TAIL16 = N_PAD - NSUB * PER16
NBLK_FULL = N // 128


def _mesh():
  return plsc.VectorSubcoreMesh(core_axis_name="c", subcore_axis_name="s")


_SC_PARAMS = pltpu.CompilerParams(use_tc_tiling_on_sc=False,
                                  needs_layout_passes=False)


def _mesh1():
  return plsc.VectorSubcoreMesh(core_axis_name="c", subcore_axis_name="s",
                                num_cores=1)


def _fill_1d(ref, val):
  n = ref.shape[0]
  v = jnp.full((HALF,), val, ref.dtype)
  for i in range(n // HALF):
    ref[pl.ds(i * HALF, HALF)] = v


def _fill_2d(ref, val):
  v = jnp.full((ref.shape[1],), val, ref.dtype)
  for i in range(ref.shape[0]):
    ref[i] = v


def _zero_slice(zbuf, dst, lo, is_last, tail):
  full = zbuf.shape[0]
  n_full, rem = divmod(PER16, full)
  for i in range(n_full):
    pltpu.sync_copy(zbuf, dst.at[pl.ds(lo + i * full, full)])
  if rem:
    pltpu.sync_copy(zbuf.at[pl.ds(0, rem)],
                    dst.at[pl.ds(lo + n_full * full, rem)])

  @pl.when(is_last)
  def _():
    base = NSUB * PER16
    n_t, rem_t = divmod(tail, full)
    for i in range(n_t):
      pltpu.sync_copy(zbuf, dst.at[pl.ds(base + i * full, full)])
    if rem_t:
      pltpu.sync_copy(zbuf.at[pl.ds(0, rem_t)],
                      dst.at[pl.ds(base + n_t * full, rem_t)])


def _copy_slice(src, dst, lo, is_last, tail):
  pltpu.sync_copy(src.at[pl.ds(lo, PER16)], dst.at[pl.ds(lo, PER16)])

  @pl.when(is_last)
  def _():
    pltpu.sync_copy(src.at[pl.ds(NSUB * PER16, tail)],
                    dst.at[pl.ds(NSUB * PER16, tail)])


def _deg_kernel_body(e4_hbm, degp_hbm, idx_v, ones_v, zbuf_v, acc_sh,
                     isem, ssem):
  c = lax.axis_index("c")
  s = lax.axis_index("s")
  w = s * NSC + c

  _fill_1d(zbuf_v, 0.0)
  lo = s * PER16
  _zero_slice(zbuf_v, acc_sh, lo, s == NSUB - 1, TAIL16)
  _fill_1d(ones_v, 1.0)
  plsc.subcore_barrier()

  sl = (NSUP * w) // 32
  sh = (NSUP * (w + 1)) // 32
  nsup = 100

  def fire_idx(k, q):
    @pl.when((k >= 0) & (sl + k < sh))
    def _():
      pltpu.async_copy(e4_hbm.at[1, sl + k], idx_v.at[q], isem)

  def wait_idx(k, q):
    @pl.when((k >= 0) & (sl + k < sh))
    def _():
      pltpu.make_async_copy(e4_hbm.at[1, sl + k], idx_v.at[q], isem).wait()

  def fire_scatter(k, q):
    @pl.when((k >= 0) & (sl + k < sh))
    def _():
      pltpu.async_copy(ones_v, acc_sh.at[idx_v.at[q, 0]], ssem, add=True)

  def wait_scatter(k, q):
    @pl.when((k >= 0) & (sl + k < sh))
    def _():
      pltpu.make_async_copy(ones_v, acc_sh.at[idx_v.at[q, 0]], ssem).wait()

  fire_idx(0, 0)

  def sbody(kk, _):
    for u in range(4):
      k = kk * 4 + u
      wait_idx(k, u)
      wait_scatter(k - 2, (u - 2) % 4)
      fire_idx(k + 1, (u + 1) % 4)
      fire_scatter(k, u)
    return 0

  lax.fori_loop(0, nsup // 4, sbody, 0)
  wait_scatter(nsup - 2, (nsup - 2) % 4)
  wait_scatter(nsup - 1, (nsup - 1) % 4)
  plsc.subcore_barrier()

  _copy_slice(acc_sh, degp_hbm.at[c], lo, s == NSUB - 1, TAIL16)


def _deg_call(e4):
  kern = pl.kernel(
      _deg_kernel_body,
      out_type=jax.ShapeDtypeStruct((NSC, N_PAD), jnp.float32),
      mesh=_mesh(),
      compiler_params=_SC_PARAMS,
      scratch_types=[
          pltpu.VMEM((4, 1, CHUNK), jnp.int32),
          pltpu.VMEM((CHUNK,), jnp.float32),
          pltpu.VMEM((1024,), jnp.float32),
          pltpu.VMEM_SHARED((N_PAD,), jnp.float32),
          pltpu.SemaphoreType.DMA,
          pltpu.SemaphoreType.DMA,
      ],
  )
  return kern(e4)


def _proj_body(ids_ref, x_ref, degp_ref, emb_ref, wp_ref, bp_ref, wg_ref,
               g_ref, gbf_ref):
  wg = wg_ref[...]
  a = jnp.dot(wp_ref[:IN_DIM, :], wg, preferred_element_type=jnp.float32)
  lut = jnp.dot(
      jnp.dot(emb_ref[...], wp_ref[IN_DIM:, :],
              preferred_element_type=jnp.float32) + bp_ref[...][None, :],
      wg, preferred_element_type=jnp.float32)
  ids = ids_ref[0]
  onehot = (ids == lax.broadcasted_iota(jnp.int32, (1, VOCAB), 1)
            ).astype(jnp.float32)
  hg = (jnp.dot(x_ref[...], a, preferred_element_type=jnp.float32)
        + jnp.dot(onehot, lut, preferred_element_type=jnp.float32))
  deg = degp_ref[0, 0] + degp_ref[1, 0] + 1.0
  g = hg * lax.rsqrt(deg)
  g_ref[0] = g[:, :HALF]
  g_ref[1] = g[:, HALF:]
  r = lax.broadcasted_iota(jnp.int32, (EMB, 1), 0)
  cc = lax.broadcasted_iota(jnp.int32, (1, EMB), 1)
  dest = jnp.where(r < HALF, 2 * r, 2 * r - (EMB - 1))
  pmat = (cc == dest).astype(jnp.float32)
  gp = jnp.dot(g, pmat, preferred_element_type=jnp.float32)
  gbf_ref[...] = gp.astype(jnp.bfloat16)


def _proj_call(ids3, x, degp4, emb, wp, bp, wg):
  blk = 4000
  grid = N // blk
  return pl.pallas_call(
      _proj_body,
      grid=(grid,),
      in_specs=[
          pl.BlockSpec((1, blk, 1), lambda i: (i, 0, 0)),
          pl.BlockSpec((blk, IN_DIM), lambda i: (i, 0)),
          pl.BlockSpec((2, 1, blk, 1), lambda i: (0, i, 0, 0)),
          pl.BlockSpec((VOCAB, EMB), lambda i: (0, 0)),
          pl.BlockSpec((IN_DIM + EMB, EMB), lambda i: (0, 0)),
          pl.BlockSpec((EMB,), lambda i: (0,)),
          pl.BlockSpec((EMB, EMB), lambda i: (0, 0)),
      ],
      out_specs=[
          pl.BlockSpec((2, blk, HALF), lambda i: (0, i, 0)),
          pl.BlockSpec((blk, EMB), lambda i: (i, 0)),
      ],
      out_shape=[
          jax.ShapeDtypeStruct((2, N, HALF), jnp.float32),
          jax.ShapeDtypeStruct((N, EMB), jnp.bfloat16),
      ],
  )(ids3, x, degp4, emb, wp, bp, wg)


def _agg_kernel_body(e4_hbm, g_hbm, acc_hbm,
                     idxs_v, idxd_v, rows_v, zbuf_v, acc_sh,
                     isem, gsem, ssem):
  s = lax.axis_index("s")

  _fill_2d(zbuf_v, 0.0)
  lo = s * PER16
  _zero_slice(zbuf_v, acc_sh, lo, s == NSUB - 1, N - NSUB * PER16)
  plsc.subcore_barrier()

  sl = (NSUP * s) // NSUB
  sh = (NSUP * (s + 1)) // NSUB
  nsup = 204

  def fire_idx(k, q):
    @pl.when((k >= 0) & (sl + k < sh))
    def _():
      pltpu.async_copy(e4_hbm.at[0, sl + k], idxs_v.at[q], isem)
      pltpu.async_copy(e4_hbm.at[1, sl + k], idxd_v.at[q], isem)

  def wait_idx(k, q):
    @pl.when((k >= 0) & (sl + k < sh))
    def _():
      pltpu.make_async_copy(e4_hbm.at[0, sl + k], idxs_v.at[q], isem).wait()
      pltpu.make_async_copy(e4_hbm.at[1, sl + k], idxd_v.at[q], isem).wait()

  def fire_gather(k, p, q):
    @pl.when((k >= 0) & (sl + k < sh))
    def _():
      pltpu.async_copy(g_hbm.at[idxs_v.at[q, 0]], rows_v.at[p], gsem)

  def wait_gather(k, p, q):
    @pl.when((k >= 0) & (sl + k < sh))
    def _():
      pltpu.make_async_copy(g_hbm.at[idxs_v.at[q, 0]], rows_v.at[p],
                            gsem).wait()

  def fire_scatter(k, p, q):
    @pl.when((k >= 0) & (sl + k < sh))
    def _():
      pltpu.async_copy(rows_v.at[p], acc_sh.at[idxd_v.at[q, 0]], ssem,
                       add=True)

  def wait_scatter(k, p, q):
    @pl.when((k >= 0) & (sl + k < sh))
    def _():
      pltpu.make_async_copy(rows_v.at[p], acc_sh.at[idxd_v.at[q, 0]],
                            ssem).wait()

  fire_idx(0, 0)
  fire_idx(1, 1)
  wait_idx(0, 0)
  fire_gather(0, 0, 0)

  def sbody(kk, _):
    for u in range(12):
      k = kk * 12 + u
      p = u % 3
      q = u % 4
      wait_idx(k + 1, (u + 1) % 4)
      wait_scatter(k - 2, (u - 2) % 3, (u - 2) % 4)
      fire_gather(k + 1, (u + 1) % 3, (u + 1) % 4)
      fire_idx(k + 2, (u + 2) % 4)
      wait_gather(k, p, q)
      fire_scatter(k, p, q)
    return 0

  lax.fori_loop(0, nsup // 12, sbody, 0)
  wait_scatter(nsup - 2, (nsup - 2) % 3, (nsup - 2) % 4)
  wait_scatter(nsup - 1, (nsup - 1) % 3, (nsup - 1) % 4)
  plsc.subcore_barrier()

  _copy_slice(acc_sh, acc_hbm, lo, s == NSUB - 1, N - NSUB * PER16)


def _agg_call(e4, gbf):
  kern = pl.kernel(
      _agg_kernel_body,
      out_type=jax.ShapeDtypeStruct((N, EMB), jnp.bfloat16),
      mesh=_mesh1(),
      compiler_params=_SC_PARAMS,
      scratch_types=[
          pltpu.VMEM((4, 1, CHUNK), jnp.int32),
          pltpu.VMEM((4, 1, CHUNK), jnp.int32),
          pltpu.VMEM((3, CHUNK, EMB), jnp.bfloat16),
          pltpu.VMEM((64, EMB), jnp.bfloat16),
          pltpu.VMEM_SHARED((N, EMB), jnp.bfloat16),
          pltpu.SemaphoreType.DMA,
          pltpu.SemaphoreType.DMA,
          pltpu.SemaphoreType.DMA,
      ],
  )
  return kern(e4, gbf)


def _epi_kernel_body(acc_hbm, g_hbm, degp_hbm, batch_hbm, bgcn_hbm,
                     partials_hbm,
                     abf_v, q0_v, q1_v, p0_v, p1_v, bt_v, bias_v,
                     seg_v):
  c = lax.axis_index("c")
  s = lax.axis_index("s")
  w = s * NSC + c
  cb = abf_v.shape[0]

  pltpu.sync_copy(bgcn_hbm, bias_v)
  b0 = bias_v[pl.ds(0, HALF)]
  b1 = bias_v[pl.ds(HALF, HALF)]

  zrow = jnp.zeros((HALF,), jnp.float32)
  for gi in range((NUM_GRAPHS + 1) * EMB // HALF):
    seg_v[pl.ds(gi * HALF, HALF)] = zrow

  lane = lax.broadcasted_iota(jnp.int32, (HALF,), 0)

  hi_mask = jnp.full((HALF,), -65536, jnp.int32)

  def do_chunk(base, size, fsize):
    pltpu.sync_copy(acc_hbm.at[pl.ds(base, size)], abf_v.at[pl.ds(0, size)])
    pltpu.sync_copy(g_hbm.at[0, pl.ds(base, size)], q0_v.at[pl.ds(0, size)])
    pltpu.sync_copy(g_hbm.at[1, pl.ds(base, size)], q1_v.at[pl.ds(0, size)])
    pltpu.sync_copy(degp_hbm.at[0, pl.ds(base, fsize)],
                    p0_v.at[pl.ds(0, fsize)])
    pltpu.sync_copy(degp_hbm.at[1, pl.ds(base, fsize)],
                    p1_v.at[pl.ds(0, fsize)])
    pltpu.sync_copy(batch_hbm.at[pl.ds(base, fsize)], bt_v.at[pl.ds(0, fsize)])

    def group_body(gi, _):
      r0 = gi * HALF
      deg = p0_v[pl.ds(r0, HALF)] + p1_v[pl.ds(r0, HALF)] + 1.0
      ib = plsc.bitcast(deg, jnp.int32)
      y = plsc.bitcast(0x5F3759DF - lax.shift_right_arithmetic(ib, 1),
                       jnp.float32)
      for _ in range(3):
        y = y * (1.5 - 0.5 * deg * y * y)
      bt = bt_v[pl.ds(r0, HALF)]
      for j in range(HALF):
        r = r0 + j
        dj = y[j]
        base_ix = bt[j] * EMB + lane
        u = plsc.bitcast(abf_v[r], jnp.int32)
        a0 = plsc.bitcast(lax.shift_left(u, 16), jnp.float32)
        a1 = plsc.bitcast(u & hi_mask, jnp.float32)
        h0 = jnp.maximum((a0 + q0_v[r]) * dj + b0, 0.0)
        h1 = jnp.maximum((a1 + q1_v[r]) * dj + b1, 0.0)
        m0 = plsc.load_gather(seg_v, [base_ix])
        plsc.store_scatter(seg_v, [base_ix], jnp.maximum(m0, h0))
        m1 = plsc.load_gather(seg_v, [base_ix + HALF])
        plsc.store_scatter(seg_v, [base_ix + HALF], jnp.maximum(m1, h1))
      return 0

    lax.fori_loop(0, size // HALF, group_body, 0)

  lo_b = (NBLK_FULL * w) // 32
  hi_b = (NBLK_FULL * (w + 1)) // 32
  base = lo_b * 128
  span = (hi_b - lo_b) * 128
  n_cb = span // cb

  def cbody(i, _):
    do_chunk(base + i * cb, cb, cb)
    return 0

  lax.fori_loop(0, n_cb, cbody, 0)
  rem = span - n_cb * cb
  base2 = base + n_cb * cb
  n128 = rem // 128

  def cbody128(i, _):
    do_chunk(base2 + i * 128, 128, 128)
    return 0

  lax.fori_loop(0, n128, cbody128, 0)

  @pl.when(w == 31)
  def _():
    do_chunk(NBLK_FULL * 128, N - NBLK_FULL * 128, 128)

  pltpu.sync_copy(seg_v, partials_hbm.at[w, 0])


def _epi_call(acc, g, degp, batch, b_gcn):
  cb = 512
  kern = pl.kernel(
      _epi_kernel_body,
      out_type=jax.ShapeDtypeStruct((32, 1, (NUM_GRAPHS + 1) * EMB),
                                    jnp.float32),
      mesh=_mesh(),
      compiler_params=_SC_PARAMS,
      scratch_types=[
          pltpu.VMEM((cb, EMB), jnp.bfloat16),
          pltpu.VMEM((cb, HALF), jnp.float32),
          pltpu.VMEM((cb, HALF), jnp.float32),
          pltpu.VMEM((cb,), jnp.float32),
          pltpu.VMEM((cb,), jnp.float32),
          pltpu.VMEM((cb,), jnp.int32),
          pltpu.VMEM((128,), jnp.float32),
          pltpu.VMEM(((NUM_GRAPHS + 1) * EMB,), jnp.float32),
      ],
  )
  return kern(acc, g, degp, batch, b_gcn)


def _head_body(part_ref, wc_ref, bc_ref, out_ref):
  rep = jnp.max(part_ref[...], axis=0)
  out_ref[...] = (jnp.dot(rep, wc_ref[...],
                          preferred_element_type=jnp.float32)
                  + bc_ref[...][None, :])


def _head_call(partials3, w_cls, b_cls):
  return pl.pallas_call(
      _head_body,
      out_shape=jax.ShapeDtypeStruct((NUM_GRAPHS, NUM_CLASSES), jnp.float32),
  )(partials3, w_cls, b_cls)


def kernel(x, node_type_ids, edge_index, batch, emb_table, W_proj, b_proj,
           W_gcn, b_gcn, W_cls, b_cls):
  e4 = edge_index.reshape(2, NSUP, 1, CHUNK)
  degp = _deg_call(e4)
  ids3 = node_type_ids.reshape(N // 4000, 4000, 1)
  degp4 = degp[:, :N].reshape(2, N // 4000, 4000, 1)
  g, gbf = _proj_call(ids3, x, degp4, emb_table, W_proj, b_proj, W_gcn)
  acc = _agg_call(e4, gbf)
  batch_pad = jnp.concatenate(
      [batch, jnp.full((N_PAD - N,), NUM_GRAPHS, jnp.int32)])
  bgcn_pad = jnp.pad(b_gcn, (0, 128 - EMB))
  partials = _epi_call(acc, g, degp, batch_pad, bgcn_pad)
  partials3 = partials.reshape(32, NUM_GRAPHS + 1, EMB)[:, :NUM_GRAPHS, :]
  return _head_call(partials3, W_cls, b_cls)
